# baseline (speedup 1.0000x reference)
import jax
import jax.numpy as jnp
from jax import lax
from jax.experimental import pallas as pl
from jax.experimental.pallas import tpu as pltpu
from jax.experimental.pallas import tpu_sc as plsc

_B, _L, _D = 4096, 200, 32
_V = 1000000
_N = _B * _L
_W = 512
_CB = _B // _W

_TA_W = 1024
_LB = 8


def _unpack_table_tc(tbl_t):
    def body(x_ref, o_ref):
        x = x_ref[...]
        o_ref[...] = (
            x.reshape(_D, _TA_W // 4, 4).transpose(1, 2, 0).reshape(_TA_W // 4, 128)
        )

    grid = (_V + _TA_W - 1) // _TA_W
    return pl.pallas_call(
        body,
        grid=(grid,),
        in_specs=[pl.BlockSpec((_D, _TA_W), lambda c: (0, c))],
        out_specs=pl.BlockSpec((_TA_W // 4, 128), lambda c: (c, 0)),
        out_shape=jax.ShapeDtypeStruct((_V // 4, 128), jnp.float32),
    )(tbl_t)


def _gather_sc(table_lin, idx):
    mesh = plsc.VectorSubcoreMesh(core_axis_name="c", subcore_axis_name="s")

    @pl.kernel(
        out_type=jax.ShapeDtypeStruct((_N, _D), jnp.float32),
        mesh=mesh,
        compiler_params=pltpu.CompilerParams(use_tc_tiling_on_sc=False),
    )
    def _embed(table_hbm, idx_hbm, out_hbm):
        def body(i_vmem, o_vmem):
            pltpu.sync_copy(table_hbm.at[i_vmem], o_vmem)

        pltpu.emit_pipeline(
            body,
            grid=(_N // _W,),
            in_specs=[pl.BlockSpec((_W,), index_map=lambda i: (i,))],
            out_specs=[pl.BlockSpec((_W, _D), index_map=lambda i: (i, 0))],
            core_axis_name=("c", "s"),
            dimension_semantics=(pltpu.PARALLEL,),
        )(idx_hbm, out_hbm)

    return _embed(table_lin, idx)


def _transpose_tc(g3):
    def body(x_ref, o_ref):
        c = pl.program_id(1)
        x = x_ref[...]
        t = (
            x.reshape(_LB, _W // 4, 4, _D).transpose(0, 3, 1, 2).reshape(_LB, _D, _W)
        )
        b_iota = lax.broadcasted_iota(jnp.int32, (_LB, _D, _W), 2)
        first = jnp.logical_and(c == 0, b_iota == 0)
        o_ref[...] = jnp.where(first, jnp.float32(0), t)

    return pl.pallas_call(
        body,
        grid=(_L // _LB, _CB),
        in_specs=[pl.BlockSpec((_LB, _W // 4, 128), lambda l, c: (l, c, 0))],
        out_specs=pl.BlockSpec((_LB, _D, _W), lambda l, c: (l, 0, c)),
        out_shape=jax.ShapeDtypeStruct((_L, _D, _B), jnp.float32),
    )(g3)


def kernel(y, table):
    yt = y.T.astype(jnp.int32)
    idx2 = jnp.pad(yt[:, :-1], ((0, 0), (1, 0)))
    idx2 = idx2.reshape(_N)
    tbl_packed = _unpack_table_tc(table.T)
    table_lin = tbl_packed.reshape(_V, _D)
    g2 = _gather_sc(table_lin, idx2)
    g3 = g2.reshape(_L, _B * _D // 128, 128)
    p = _transpose_tc(g3)
    return jnp.transpose(p, (2, 0, 1))


# speedup vs baseline: 5.8450x; 5.8450x over previous
import jax
import jax.numpy as jnp
from jax import lax
from jax.experimental import pallas as pl
from jax.experimental.pallas import tpu as pltpu
from jax.experimental.pallas import tpu_sc as plsc

_B, _L, _D = 4096, 200, 32
_V = 1000000
_N = _B * _L
_W = 512
_CB = _B // _W

_TA_W = 512
_TA_SUB = 8


_DOT = dict(precision=jax.lax.Precision.DEFAULT,
            preferred_element_type=jnp.float32)


def _unpack_table_tc(tbl_t, spread, ident):
    def body(x_ref, s_ref, i_ref, o_ref):
        for s in range(_TA_SUB):
            x = x_ref[:, _TA_W * s:_TA_W * (s + 1)]
            parts = []
            for j in range(4):
                parts.append(jax.lax.dot_general(
                    x, s_ref[j], (((1,), (0,)), ((), ())), **_DOT))
            out_t = jnp.concatenate(parts, axis=0)
            o_ref[_TA_W // 4 * s:_TA_W // 4 * (s + 1), :] = jax.lax.dot_general(
                i_ref[...], out_t, (((0,), (1,)), ((), ())), **_DOT)

    big = _TA_W * _TA_SUB
    grid = (_V + big - 1) // big
    return pl.pallas_call(
        body,
        grid=(grid,),
        in_specs=[
            pl.BlockSpec((_D, big), lambda c: (0, c)),
            pl.BlockSpec((4, _TA_W, _TA_W // 4), lambda c: (0, 0, 0)),
            pl.BlockSpec((128, 128), lambda c: (0, 0)),
        ],
        out_specs=pl.BlockSpec((big // 4, 128), lambda c: (c, 0)),
        out_shape=jax.ShapeDtypeStruct((_V // 4, 128), jnp.float32),
    )(tbl_t, spread, ident)


def _gather_sc(table_lin, idx):
    mesh = plsc.VectorSubcoreMesh(core_axis_name="c", subcore_axis_name="s")

    @pl.kernel(
        out_type=jax.ShapeDtypeStruct((_N, _D), jnp.float32),
        mesh=mesh,
        compiler_params=pltpu.CompilerParams(use_tc_tiling_on_sc=False),
    )
    def _embed(table_hbm, idx_hbm, out_hbm):
        def body(i_vmem, o_vmem):
            pltpu.sync_copy(table_hbm.at[i_vmem], o_vmem)

        pltpu.emit_pipeline(
            body,
            grid=(_N // _W,),
            in_specs=[pl.BlockSpec((_W,), index_map=lambda i: (i,))],
            out_specs=[pl.BlockSpec((_W, _D), index_map=lambda i: (i, 0))],
            core_axis_name=("c", "s"),
            dimension_semantics=(pltpu.PARALLEL,),
        )(idx_hbm, out_hbm)

    return _embed(table_lin, idx)


def _transpose_tc(g3, spread2, ident):
    def body(x_ref, s_ref, i_ref, o_ref):
        for c in range(_CB):
            x = x_ref[0, 128 * c:128 * (c + 1), :]
            xt = jax.lax.dot_general(
                i_ref[...], x, (((0,), (1,)), ((), ())), **_DOT)
            o = jnp.zeros((_D, _W), jnp.float32)
            for j in range(4):
                aj = xt[_D * j:_D * (j + 1), :]
                o = o + jax.lax.dot_general(
                    aj, s_ref[j], (((1,), (0,)), ((), ())), **_DOT)
            if c == 0:
                b_iota = lax.broadcasted_iota(jnp.int32, (_D, _W), 1)
                o = jnp.where(b_iota == 0, jnp.float32(0), o)
            o_ref[0, :, _W * c:_W * (c + 1)] = o

    return pl.pallas_call(
        body,
        grid=(_L,),
        in_specs=[
            pl.BlockSpec((1, _B // 4, 128), lambda l: (l, 0, 0)),
            pl.BlockSpec((4, 128, _W), lambda l: (0, 0, 0)),
            pl.BlockSpec((128, 128), lambda l: (0, 0)),
        ],
        out_specs=pl.BlockSpec((1, _D, _B), lambda l: (l, 0, 0)),
        out_shape=jax.ShapeDtypeStruct((_L, _D, _B), jnp.float32),
    )(g3, spread2, ident)


def _spread(n, m):
    i_ar = jnp.arange(n)[None, :, None]
    r_ar = jnp.arange(m)[None, None, :]
    j_ar = jnp.arange(4)[:, None, None]
    return (i_ar == 4 * r_ar + j_ar).astype(jnp.float32)


def kernel(y, table):
    yt = y.T.astype(jnp.int32)
    idx2 = jnp.pad(yt[:, :-1], ((0, 0), (1, 0)))
    idx2 = idx2.reshape(_N)
    ident = jnp.eye(128, dtype=jnp.float32)
    sp_a = _spread(_TA_W, _TA_W // 4)
    sp_b = _spread(_W, _W // 4).transpose(0, 2, 1)
    tbl_packed = _unpack_table_tc(table.T, sp_a, ident)
    table_lin = tbl_packed.reshape(_V, _D)
    g2 = _gather_sc(table_lin, idx2)
    g3 = g2.reshape(_L, _B * _D // 128, 128)
    p = _transpose_tc(g3, sp_b, ident)
    return jnp.transpose(p, (2, 0, 1))


# speedup vs baseline: 8.6407x; 1.4783x over previous
import jax
import jax.numpy as jnp
from jax import lax
from jax.experimental import pallas as pl
from jax.experimental.pallas import tpu as pltpu
from jax.experimental.pallas import tpu_sc as plsc

_B, _L, _D = 4096, 200, 32
_V = 1000000
_N = _B * _L
_W = 512
_CB = _B // _W

_Q = 262144
_TA_W = 512
_TA_SUB = 4


_DOT = dict(precision=jax.lax.Precision.DEFAULT,
            preferred_element_type=jnp.float32)


def _unpack_table_tc(tbl_t, ident):
    def body(x0, x1, x2, x3, i_ref, o_ref):
        for s in range(_TA_SUB):
            sl = slice(_TA_W * s, _TA_W * (s + 1))
            xc = jnp.concatenate(
                [x0[:, sl], x1[:, sl], x2[:, sl], x3[:, sl]], axis=0)
            o_ref[sl, :] = jax.lax.dot_general(
                xc, i_ref[...], (((0,), (0,)), ((), ())), **_DOT)

    big = _TA_W * _TA_SUB
    last = _V // big
    xspec = [
        pl.BlockSpec(
            (_D, big),
            (lambda c, u=u: (0, jnp.minimum(c + u * (_Q // big), last))))
        for u in range(4)
    ]
    return pl.pallas_call(
        body,
        grid=(_Q // big,),
        in_specs=xspec + [pl.BlockSpec((128, 128), lambda c: (0, 0))],
        out_specs=pl.BlockSpec((big, 128), lambda c: (c, 0)),
        out_shape=jax.ShapeDtypeStruct((_Q, 128), jnp.float32),
    )(tbl_t, tbl_t, tbl_t, tbl_t, ident)


def _gather_sc(table_lin, idx):
    mesh = plsc.VectorSubcoreMesh(core_axis_name="c", subcore_axis_name="s")

    @pl.kernel(
        out_type=jax.ShapeDtypeStruct((_N, _D), jnp.float32),
        mesh=mesh,
        compiler_params=pltpu.CompilerParams(use_tc_tiling_on_sc=False),
    )
    def _embed(table_hbm, idx_hbm, out_hbm):
        def body(i_vmem, o_vmem):
            pltpu.sync_copy(table_hbm.at[i_vmem], o_vmem)

        pltpu.emit_pipeline(
            body,
            grid=(_N // _W,),
            in_specs=[pl.BlockSpec((_W,), index_map=lambda i: (i,))],
            out_specs=[pl.BlockSpec((_W, _D), index_map=lambda i: (i, 0))],
            core_axis_name=("c", "s"),
            dimension_semantics=(pltpu.PARALLEL,),
        )(idx_hbm, out_hbm)

    return _embed(table_lin, idx)


def _transpose_tc(g3, spread2, ident):
    def body(x_ref, s_ref, i_ref, o_ref):
        for c in range(_CB):
            x = x_ref[0, 128 * c:128 * (c + 1), :]
            xt = jax.lax.dot_general(
                i_ref[...], x, (((0,), (1,)), ((), ())), **_DOT)
            o = jnp.zeros((_D, _W), jnp.float32)
            for j in range(4):
                aj = xt[_D * j:_D * (j + 1), :]
                o = o + jax.lax.dot_general(
                    aj, s_ref[j], (((1,), (0,)), ((), ())), **_DOT)
            if c == 0:
                b_iota = lax.broadcasted_iota(jnp.int32, (_D, _W), 1)
                o = jnp.where(b_iota == 0, jnp.float32(0), o)
            o_ref[0, :, _W * c:_W * (c + 1)] = o

    return pl.pallas_call(
        body,
        grid=(_L,),
        in_specs=[
            pl.BlockSpec((1, _B // 4, 128), lambda l: (l, 0, 0)),
            pl.BlockSpec((4, 128, _W), lambda l: (0, 0, 0)),
            pl.BlockSpec((128, 128), lambda l: (0, 0)),
        ],
        out_specs=pl.BlockSpec((1, _D, _B), lambda l: (l, 0, 0)),
        out_shape=jax.ShapeDtypeStruct((_L, _D, _B), jnp.float32),
    )(g3, spread2, ident)


def _spread(n, m):
    i_ar = jnp.arange(n)[None, :, None]
    r_ar = jnp.arange(m)[None, None, :]
    j_ar = jnp.arange(4)[:, None, None]
    return (i_ar == 4 * r_ar + j_ar).astype(jnp.float32)


def kernel(y, table):
    yt = y.T.astype(jnp.int32)
    idx2 = jnp.pad(yt[:, :-1], ((0, 0), (1, 0)))
    idx2 = idx2.reshape(_N)
    idx2 = 4 * (idx2 % _Q) + idx2 // _Q

    ident = jnp.eye(128, dtype=jnp.float32)
    sp_b = _spread(_W, _W // 4).transpose(0, 2, 1)
    tbl_packed = _unpack_table_tc(table.T, ident)
    table_lin = tbl_packed.reshape(4 * _Q, _D)
    g2 = _gather_sc(table_lin, idx2)
    g3 = g2.reshape(_L, _B * _D // 128, 128)
    p = _transpose_tc(g3, sp_b, ident)
    return jnp.transpose(p, (2, 0, 1))
